# baseline (speedup 1.0000x reference)
import functools

import jax
import jax.numpy as jnp
from jax import lax
from jax.experimental import pallas as pl
from jax.experimental.pallas import tpu as pltpu
from jax.experimental.pallas import tpu_sc as plsc

NC = 2
NS = 16
NW = NC * NS
CHUNK = 128
NB = 4


def _node_pad(n):
    per = -(-(n + 1) // (NS * 8)) * 8
    return NS * per, per


DEGW = 8


def _make_deg(n, cpw):
    npad, rows_per_tile = _node_pad(n)
    mesh = plsc.VectorSubcoreMesh(
        core_axis_name="c", subcore_axis_name="s", num_cores=NC, num_subcores=NS)

    @functools.partial(
        pl.kernel,
        out_type=jax.ShapeDtypeStruct((NC * npad, DEGW), jnp.float32),
        mesh=mesh,
        compiler_params=pltpu.CompilerParams(use_tc_tiling_on_sc=False),
        scratch_types=[
            pltpu.VMEM((cpw, CHUNK), jnp.int32),
            pltpu.VMEM((CHUNK, DEGW), jnp.float32),
            pltpu.VMEM_SHARED((npad, DEGW), jnp.float32),
        ],
    )
    def deg_kernel(dsts_hbm, ones_hbm, zeros_hbm, out_hbm, dst_v, ones_v, acc_sh):
        cid = lax.axis_index("c")
        sid = lax.axis_index("s")
        wid = sid * NC + cid
        r0 = sid * rows_per_tile
        pltpu.sync_copy(zeros_hbm.at[pl.ds(r0, rows_per_tile)],
                        acc_sh.at[pl.ds(r0, rows_per_tile)])
        pltpu.sync_copy(dsts_hbm.at[pl.ds(wid * cpw, cpw)], dst_v)
        pltpu.sync_copy(ones_hbm, ones_v)
        plsc.subcore_barrier()

        def body(c, carry):
            pltpu.sync_copy(ones_v, acc_sh.at[dst_v.at[c]], add=True)
            return carry

        lax.fori_loop(0, cpw, body, 0)
        plsc.subcore_barrier()
        pltpu.sync_copy(acc_sh.at[pl.ds(r0, rows_per_tile)],
                        out_hbm.at[pl.ds(cid * npad + r0, rows_per_tile)])

    return deg_kernel, npad


def _make_agg(n, cpw0, cpw1, width):
    npad, rows_per_tile = _node_pad(n)
    mesh = plsc.VectorSubcoreMesh(
        core_axis_name="c", subcore_axis_name="s", num_cores=NC, num_subcores=NS)
    assert cpw0 % NB == 0 and cpw1 % NB == 0 and cpw0 % 8 == 0 and cpw1 % 8 == 0

    @functools.partial(
        pl.kernel,
        out_type=jax.ShapeDtypeStruct((npad, width), jnp.float32),
        mesh=mesh,
        compiler_params=pltpu.CompilerParams(use_tc_tiling_on_sc=False),
        scratch_types=[
            pltpu.VMEM((cpw0, CHUNK), jnp.int32),
            pltpu.VMEM((cpw0, CHUNK), jnp.int32),
            pltpu.VMEM((NB, CHUNK, width), jnp.float32),
            pltpu.VMEM_SHARED((npad, width), jnp.float32),
            pltpu.SemaphoreType.DMA((NB,)),
            pltpu.SemaphoreType.DMA((NB,)),
        ],
    )
    def agg_kernel(hs_hbm, srcs_hbm, dsts_hbm, zeros_hbm, out_hbm,
                   src_v, dst_v, rows_v, acc_sh, gsem, ssem):
        cid = lax.axis_index("c")
        sid = lax.axis_index("s")
        r0 = sid * rows_per_tile

        def run(my_cpw, c0):
            pltpu.sync_copy(zeros_hbm.at[pl.ds(r0, rows_per_tile)],
                            acc_sh.at[pl.ds(r0, rows_per_tile)])
            pltpu.sync_copy(srcs_hbm.at[pl.ds(c0, my_cpw)],
                            src_v.at[pl.ds(0, my_cpw)])
            pltpu.sync_copy(dsts_hbm.at[pl.ds(c0, my_cpw)],
                            dst_v.at[pl.ds(0, my_cpw)])
            plsc.subcore_barrier()

            for b in range(NB):
                pltpu.async_copy(hs_hbm.at[src_v.at[b]], rows_v.at[b],
                                 gsem.at[b])

            def body(i, carry):
                for b in range(NB):
                    s = i * NB + b
                    pltpu.make_async_copy(
                        hs_hbm.at[src_v.at[s]], rows_v.at[b], gsem.at[b]).wait()
                    pltpu.async_copy(
                        rows_v.at[b], acc_sh.at[dst_v.at[s]], ssem.at[b],
                        add=True)
                    nxt = s + NB

                    @pl.when(nxt < my_cpw)
                    def _():
                        pltpu.make_async_copy(
                            rows_v.at[b], acc_sh.at[dst_v.at[s]],
                            ssem.at[b]).wait()
                        pltpu.async_copy(
                            hs_hbm.at[src_v.at[nxt]], rows_v.at[b], gsem.at[b])

                return carry

            lax.fori_loop(0, my_cpw // NB, body, 0)
            for b in range(NB):
                pltpu.make_async_copy(
                    rows_v.at[b], acc_sh.at[dst_v.at[my_cpw - NB + b]],
                    ssem.at[b]).wait()

        @pl.when(cid == 0)
        def _():
            run(cpw0, sid * cpw0)
            plsc.subcore_barrier()
            pltpu.sync_copy(acc_sh.at[pl.ds(r0, rows_per_tile)],
                            out_hbm.at[pl.ds(r0, rows_per_tile)])

    return agg_kernel, npad


def _dinv_of(degp_ref):
    npad = degp_ref.shape[0] // NC
    deg = degp_ref[:npad, 0:1] + degp_ref[npad:, 0:1] + 1.0
    return lax.rsqrt(deg)


def _pre_body(x_ref, w_ref, degp_ref, hs_ref):
    n = x_ref.shape[0]
    npad = hs_ref.shape[0]
    dinv = _dinv_of(degp_ref)
    h = jnp.dot(x_ref[...], w_ref[...], preferred_element_type=jnp.float32)
    hs_ref[...] = jnp.concatenate(
        [h * dinv[:n], jnp.zeros((npad - n, h.shape[1]), h.dtype)], axis=0)


def _mid_body(agg_ref, hs1_ref, degp_ref, b1_ref, hs2_ref):
    dinv = _dinv_of(degp_ref)
    h1 = dinv * (agg_ref[...] + hs1_ref[...]) + b1_ref[...]
    hs2_ref[...] = jnp.maximum(h1, 0.0) * dinv


def _fin_body(agg_ref, hs2_ref, degp_ref, wmu_ref, bmu_ref,
              wlv_ref, blv_ref, eps_ref, z_ref, mu_ref, lv_ref):
    dinv = _dinv_of(degp_ref)
    n = eps_ref.shape[0]
    g = (dinv * (agg_ref[...] + hs2_ref[...]))[:n]
    mu = jnp.dot(g, wmu_ref[...], preferred_element_type=jnp.float32) + bmu_ref[...]
    lv = jnp.dot(g, wlv_ref[...], preferred_element_type=jnp.float32) + blv_ref[...]
    z = mu + eps_ref[...] * jnp.exp(0.5 * lv)
    z_ref[...] = z
    mu_ref[...] = mu
    lv_ref[...] = lv


def kernel(x, W1, b1, Wmu, bmu, Wlv, blv, edge_index):
    if x.ndim == 1:
        x = x[:, None]
    n = x.shape[0]
    hid = W1.shape[1]
    lat = Wmu.shape[1]
    e = edge_index.shape[1]

    cpw = -(-(-(-e // (NW * CHUNK))) // 8) * 8
    ep = cpw * NW * CHUNK
    pad = ep - e
    src2d = jnp.concatenate(
        [edge_index[0], jnp.zeros((pad,), edge_index.dtype)]).reshape(-1, CHUNK)
    dst2d = jnp.concatenate(
        [edge_index[1], jnp.full((pad,), n, edge_index.dtype)]).reshape(-1, CHUNK)

    cpw1 = 0
    cpw0 = 2 * cpw - cpw1
    deg_kernel, npad = _make_deg(n, cpw)
    agg_kernel, _ = _make_agg(n, cpw0, cpw1, hid)
    ones8 = jnp.ones((CHUNK, DEGW), jnp.float32)
    zeros8 = jnp.zeros((npad, DEGW), jnp.float32)
    zeros2 = jnp.zeros((npad, hid), jnp.float32)

    degp = deg_kernel(dst2d, ones8, zeros8)

    hs1 = pl.pallas_call(
        _pre_body,
        out_shape=jax.ShapeDtypeStruct((npad, hid), jnp.float32),
    )(x, W1, degp)

    agg1 = agg_kernel(hs1, src2d, dst2d, zeros2)

    hs2 = pl.pallas_call(
        _mid_body,
        out_shape=jax.ShapeDtypeStruct((npad, hid), jnp.float32),
    )(agg1, hs1, degp, b1.reshape(1, -1))

    agg2 = agg_kernel(hs2, src2d, dst2d, zeros2)

    eps = jax.random.normal(jax.random.key(42), (n, lat), dtype=jnp.float32)
    z, mu, lv = pl.pallas_call(
        _fin_body,
        out_shape=(
            jax.ShapeDtypeStruct((n, lat), jnp.float32),
            jax.ShapeDtypeStruct((n, lat), jnp.float32),
            jax.ShapeDtypeStruct((n, lat), jnp.float32),
        ),
    )(agg2, hs2, degp,
      Wmu, bmu.reshape(1, -1), Wlv, blv.reshape(1, -1), eps)

    return (z, mu, lv)


# speedup vs baseline: 1.1389x; 1.1389x over previous
import functools

import jax
import jax.numpy as jnp
from jax import lax
from jax.experimental import pallas as pl
from jax.experimental.pallas import tpu as pltpu
from jax.experimental.pallas import tpu_sc as plsc

NC = 2
NS = 16
NW = NC * NS
CHUNK = 128
NB = 4


def _node_pad(n):
    per = -(-(n + 1) // (NS * 8)) * 8
    return NS * per, per


DEGW = 8


def _make_deg(n, cpw):
    npad, rows_per_tile = _node_pad(n)
    mesh = plsc.VectorSubcoreMesh(
        core_axis_name="c", subcore_axis_name="s", num_cores=NC, num_subcores=NS)

    @functools.partial(
        pl.kernel,
        out_type=jax.ShapeDtypeStruct((NC * npad, DEGW), jnp.float32),
        mesh=mesh,
        compiler_params=pltpu.CompilerParams(use_tc_tiling_on_sc=False),
        scratch_types=[
            pltpu.VMEM((cpw, CHUNK), jnp.int32),
            pltpu.VMEM((CHUNK, DEGW), jnp.float32),
            pltpu.VMEM_SHARED((npad, DEGW), jnp.float32),
        ],
    )
    def deg_kernel(dsts_hbm, ones_hbm, zeros_hbm, out_hbm, dst_v, ones_v, acc_sh):
        cid = lax.axis_index("c")
        sid = lax.axis_index("s")
        wid = sid * NC + cid
        r0 = sid * rows_per_tile
        pltpu.sync_copy(zeros_hbm.at[pl.ds(r0, rows_per_tile)],
                        acc_sh.at[pl.ds(r0, rows_per_tile)])
        pltpu.sync_copy(dsts_hbm.at[pl.ds(wid * cpw, cpw)], dst_v)
        pltpu.sync_copy(ones_hbm, ones_v)
        plsc.subcore_barrier()

        def body(c, carry):
            pltpu.sync_copy(ones_v, acc_sh.at[dst_v.at[c]], add=True)
            return carry

        lax.fori_loop(0, cpw, body, 0)
        plsc.subcore_barrier()
        pltpu.sync_copy(acc_sh.at[pl.ds(r0, rows_per_tile)],
                        out_hbm.at[pl.ds(cid * npad + r0, rows_per_tile)])

    return deg_kernel, npad


def _make_agg(n, cpw0, cpw1, width):
    npad, rows_per_tile = _node_pad(n)
    mesh = plsc.VectorSubcoreMesh(
        core_axis_name="c", subcore_axis_name="s", num_cores=NC, num_subcores=NS)
    assert cpw0 % NB == 0 and cpw1 % NB == 0 and cpw0 % 8 == 0 and cpw1 % 8 == 0

    @functools.partial(
        pl.kernel,
        out_type=jax.ShapeDtypeStruct((NC, npad, width), jnp.float32),
        mesh=mesh,
        compiler_params=pltpu.CompilerParams(use_tc_tiling_on_sc=False),
        scratch_types=[
            pltpu.VMEM((cpw0, CHUNK), jnp.int32),
            pltpu.VMEM((cpw0, CHUNK), jnp.int32),
            pltpu.VMEM((NB, CHUNK, width), jnp.float32),
            pltpu.VMEM_SHARED((npad, width), jnp.float32),
            pltpu.SemaphoreType.DMA((NB,)),
            pltpu.SemaphoreType.DMA((NB,)),
        ],
    )
    def agg_kernel(hs_hbm, srcs_hbm, dsts_hbm, zeros_hbm, out_hbm,
                   src_v, dst_v, rows_v, acc_sh, gsem, ssem):
        cid = lax.axis_index("c")
        sid = lax.axis_index("s")
        r0 = sid * rows_per_tile
        pltpu.sync_copy(zeros_hbm.at[pl.ds(r0, rows_per_tile)],
                        acc_sh.at[pl.ds(r0, rows_per_tile)])

        def run(my_cpw, c0):
            pltpu.sync_copy(srcs_hbm.at[pl.ds(c0, my_cpw)],
                            src_v.at[pl.ds(0, my_cpw)])
            pltpu.sync_copy(dsts_hbm.at[pl.ds(c0, my_cpw)],
                            dst_v.at[pl.ds(0, my_cpw)])
            plsc.subcore_barrier()

            for b in range(NB):
                pltpu.async_copy(hs_hbm.at[src_v.at[b]], rows_v.at[b],
                                 gsem.at[b])

            def body(i, carry):
                for b in range(NB):
                    s = i * NB + b
                    pltpu.make_async_copy(
                        hs_hbm.at[src_v.at[s]], rows_v.at[b], gsem.at[b]).wait()
                    pltpu.async_copy(
                        rows_v.at[b], acc_sh.at[dst_v.at[s]], ssem.at[b],
                        add=True)
                    nxt = s + NB

                    @pl.when(nxt < my_cpw)
                    def _():
                        pltpu.make_async_copy(
                            rows_v.at[b], acc_sh.at[dst_v.at[s]],
                            ssem.at[b]).wait()
                        pltpu.async_copy(
                            hs_hbm.at[src_v.at[nxt]], rows_v.at[b], gsem.at[b])

                return carry

            lax.fori_loop(0, my_cpw // NB, body, 0)
            for b in range(NB):
                pltpu.make_async_copy(
                    rows_v.at[b], acc_sh.at[dst_v.at[my_cpw - NB + b]],
                    ssem.at[b]).wait()

        @pl.when(cid == 0)
        def _():
            run(cpw0, sid * cpw0)

        @pl.when(cid == 1)
        def _():
            run(cpw1, NS * cpw0 + sid * cpw1)

        plsc.subcore_barrier()
        pltpu.sync_copy(acc_sh.at[pl.ds(r0, rows_per_tile)],
                        out_hbm.at[cid, pl.ds(r0, rows_per_tile)])

    return agg_kernel, npad


def _dinv_of(degp_ref):
    npad = degp_ref.shape[0] // NC
    deg = degp_ref[:npad, 0:1] + degp_ref[npad:, 0:1] + 1.0
    return lax.rsqrt(deg)


def _pre_body(x_ref, w_ref, degp_ref, hs_ref):
    n = x_ref.shape[0]
    npad = hs_ref.shape[0]
    dinv = _dinv_of(degp_ref)
    h = jnp.dot(x_ref[...], w_ref[...], preferred_element_type=jnp.float32)
    hs_ref[...] = jnp.concatenate(
        [h * dinv[:n], jnp.zeros((npad - n, h.shape[1]), h.dtype)], axis=0)


def _mid_body(agg_ref, hs1_ref, degp_ref, b1_ref, hs2_ref):
    dinv = _dinv_of(degp_ref)
    h1 = dinv * (agg_ref[0] + agg_ref[1] + hs1_ref[...]) + b1_ref[...]
    hs2_ref[...] = jnp.maximum(h1, 0.0) * dinv


def _fin_body(agg_ref, hs2_ref, degp_ref, wmu_ref, bmu_ref,
              wlv_ref, blv_ref, eps_ref, z_ref, mu_ref, lv_ref):
    dinv = _dinv_of(degp_ref)
    n = eps_ref.shape[0]
    g = (dinv * (agg_ref[0] + agg_ref[1] + hs2_ref[...]))[:n]
    mu = jnp.dot(g, wmu_ref[...], preferred_element_type=jnp.float32) + bmu_ref[...]
    lv = jnp.dot(g, wlv_ref[...], preferred_element_type=jnp.float32) + blv_ref[...]
    z = mu + eps_ref[...] * jnp.exp(0.5 * lv)
    z_ref[...] = z
    mu_ref[...] = mu
    lv_ref[...] = lv


def kernel(x, W1, b1, Wmu, bmu, Wlv, blv, edge_index):
    if x.ndim == 1:
        x = x[:, None]
    n = x.shape[0]
    hid = W1.shape[1]
    lat = Wmu.shape[1]
    e = edge_index.shape[1]

    cpw = -(-(-(-e // (NW * CHUNK))) // 8) * 8
    ep = cpw * NW * CHUNK
    pad = ep - e
    src2d = jnp.concatenate(
        [edge_index[0], jnp.zeros((pad,), edge_index.dtype)]).reshape(-1, CHUNK)
    dst2d = jnp.concatenate(
        [edge_index[1], jnp.full((pad,), n, edge_index.dtype)]).reshape(-1, CHUNK)

    cpw1 = 24
    cpw0 = 2 * cpw - cpw1
    deg_kernel, npad = _make_deg(n, cpw)
    agg_kernel, _ = _make_agg(n, cpw0, cpw1, hid)
    ones8 = jnp.ones((CHUNK, DEGW), jnp.float32)
    zeros8 = jnp.zeros((npad, DEGW), jnp.float32)
    zeros2 = jnp.zeros((npad, hid), jnp.float32)

    degp = deg_kernel(dst2d, ones8, zeros8)

    hs1 = pl.pallas_call(
        _pre_body,
        out_shape=jax.ShapeDtypeStruct((npad, hid), jnp.float32),
    )(x, W1, degp)

    agg1 = agg_kernel(hs1, src2d, dst2d, zeros2)

    hs2 = pl.pallas_call(
        _mid_body,
        out_shape=jax.ShapeDtypeStruct((npad, hid), jnp.float32),
    )(agg1, hs1, degp, b1.reshape(1, -1))

    agg2 = agg_kernel(hs2, src2d, dst2d, zeros2)

    eps = jax.random.normal(jax.random.key(42), (n, lat), dtype=jnp.float32)
    z, mu, lv = pl.pallas_call(
        _fin_body,
        out_shape=(
            jax.ShapeDtypeStruct((n, lat), jnp.float32),
            jax.ShapeDtypeStruct((n, lat), jnp.float32),
            jax.ShapeDtypeStruct((n, lat), jnp.float32),
        ),
    )(agg2, hs2, degp,
      Wmu, bmu.reshape(1, -1), Wlv, blv.reshape(1, -1), eps)

    return (z, mu, lv)
